# baseline (speedup 1.0000x reference)
import functools

import jax
import jax.numpy as jnp
from jax import lax
from jax.experimental import pallas as pl
from jax.experimental.pallas import tpu as pltpu
from jax.experimental.pallas import tpu_sc as plsc

B = 4096
L = 200
D = 32
VOCAB = 1000000

NC = 2
NS = 16
NW = NC * NS
SEQ_PER = B // NW

_L_STARTS = tuple(range(0, L - 16, 16)) + (L - 16,)


def _body(x_hbm, table_hbm, emb_hbm, len_hbm, idx_v, rows_v, outT_v, len_v, sem):
    wid = lax.axis_index("s") * NC + lax.axis_index("c")
    iota = lax.iota(jnp.int32, 16)
    lane0 = iota == 0
    tail_mask = iota >= 8

    def seq_body(i, carry):
        seq = wid * SEQ_PER + i
        pltpu.sync_copy(x_hbm.at[seq, pl.ds(0, 128)], idx_v.at[0])
        pltpu.sync_copy(x_hbm.at[seq, pl.ds(128, 72)], idx_v.at[1, pl.ds(0, 72)])

        cp0 = pltpu.async_copy(table_hbm.at[idx_v.at[0]], rows_v.at[pl.ds(0, 128)], sem)
        cp1 = pltpu.async_copy(
            table_hbm.at[idx_v.at[1, pl.ds(0, 72)]], rows_v.at[pl.ds(128, 72)], sem
        )

        cnt = jnp.zeros((16,), jnp.int32)
        one = jnp.ones((16,), jnp.int32)
        zero = jnp.zeros((16,), jnp.int32)
        for c in range(8):
            v = idx_v[0, pl.ds(c * 16, 16)]
            cnt = cnt + jnp.where(v != 0, one, zero)
        for c in range(4):
            v = idx_v[1, pl.ds(c * 16, 16)]
            cnt = cnt + jnp.where(v != 0, one, zero)
        v = idx_v[1, pl.ds(56, 16)]
        cnt = cnt + jnp.where((v != 0) & tail_mask, one, zero)
        total = jnp.full((16,), jnp.sum(cnt), jnp.int32)
        plsc.store_scatter(len_v, [jnp.full((16,), i, jnp.int32)], total, mask=lane0)

        cp0.wait()
        cp1.wait()

        for d in range(D):
            col = jnp.full((16,), d, jnp.int32)
            for l0 in _L_STARTS:
                vec = plsc.load_gather(rows_v, [iota + l0, col])
                outT_v[d, pl.ds(l0, 16)] = vec

        pltpu.sync_copy(outT_v, emb_hbm.at[seq])
        return carry

    lax.fori_loop(0, SEQ_PER, seq_body, 0)
    pltpu.sync_copy(len_v, len_hbm.at[pl.ds(wid * SEQ_PER, SEQ_PER)])


@jax.jit
def _run(x, table):
    mesh = plsc.VectorSubcoreMesh(core_axis_name="c", subcore_axis_name="s")
    return pl.kernel(
        _body,
        out_type=(
            jax.ShapeDtypeStruct((B, D, L), jnp.float32),
            jax.ShapeDtypeStruct((B,), jnp.int32),
        ),
        mesh=mesh,
        compiler_params=pltpu.CompilerParams(
            needs_layout_passes=False, use_tc_tiling_on_sc=False
        ),
        scratch_types=(
            pltpu.VMEM((2, 128), jnp.int32),
            pltpu.VMEM((L, D), jnp.float32),
            pltpu.VMEM((D, L), jnp.float32),
            pltpu.VMEM((SEQ_PER,), jnp.int32),
            pltpu.SemaphoreType.DMA,
        ),
    )(x, table)


def kernel(x, table):
    return _run(x, table)
